# Initial kernel scaffold; baseline (speedup 1.0000x reference)
import jax
import jax.numpy as jnp
from jax.experimental import pallas as pl


def kernel(x, token_table, pos_table):
    raise NotImplementedError("write your pallas kernel here")


# speedup vs baseline: 4.4411x; 4.4411x over previous
import functools

import jax
import jax.numpy as jnp
from jax import lax
from jax.experimental import pallas as pl
from jax.experimental.pallas import tpu as pltpu
from jax.experimental.pallas import tpu_sc as plsc

_L = 16
_GCH = 128


@functools.lru_cache(maxsize=None)
def _make_kernel(B, T, D, V):
    NC, NS = 2, 16
    NW = NC * NS
    CH = 256
    NQ = B // CH
    DV = D // _L
    NG = CH // _GCH

    mesh = plsc.VectorSubcoreMesh(core_axis_name="c", subcore_axis_name="s")

    @functools.partial(
        pl.kernel,
        out_type=jax.ShapeDtypeStruct((B * T, D), jnp.float32),
        mesh=mesh,
        scratch_types=[
            pltpu.VMEM((CH,), jnp.int32),
            pltpu.VMEM((CH, D), jnp.float32),
            pltpu.VMEM((T, D), jnp.float32),
            pltpu.VMEM((NG, _GCH), jnp.int32),
            pltpu.SemaphoreType.DMA,
            pltpu.SemaphoreType.DMA,
        ],
    )
    def kern(xt_hbm, tok_hbm, pos_hbm, out_hbm,
             idx_v, rows_v, pos_v, oidx_v, gsem, ssem):
        cid = lax.axis_index("c")
        sid = lax.axis_index("s")
        wid = sid * NC + cid

        pltpu.sync_copy(pos_hbm, pos_v)

        nt = (T - wid + NW - 1) // NW
        ntasks = nt * NQ

        def task(i, carry):
            t = wid + (i // NQ) * NW
            base_b = (i % NQ) * CH

            pltpu.sync_copy(xt_hbm.at[pl.ds(t * B + base_b, CH)], idx_v)

            cps = [
                pltpu.async_copy(
                    tok_hbm.at[idx_v.at[pl.ds(j * _GCH, _GCH)]],
                    rows_v.at[pl.ds(j * _GCH, _GCH)],
                    gsem,
                )
                for j in range(NG)
            ]
            for cp in cps:
                cp.wait()

            posv = [pos_v[t, pl.ds(k * _L, _L)] for k in range(DV)]

            def row(r, c):
                for k in range(DV):
                    sl = pl.ds(k * _L, _L)
                    rows_v[r, sl] = rows_v[r, sl] + posv[k]
                return c

            lax.fori_loop(0, CH, row, 0)

            lanes = lax.iota(jnp.int32, _L) * T
            for j in range(NG):
                for k in range(_GCH // _L):
                    c0 = (base_b + j * _GCH + k * _L) * T
                    oidx_v[j, pl.ds(k * _L, _L)] = lanes + (c0 + t)

            sps = [
                pltpu.async_copy(
                    rows_v.at[pl.ds(j * _GCH, _GCH)],
                    out_hbm.at[oidx_v.at[j]],
                    ssem,
                )
                for j in range(NG)
            ]
            for sp in sps:
                sp.wait()
            return carry

        lax.fori_loop(0, ntasks, task, 0)

    return kern


def kernel(x, token_table, pos_table):
    B, T = x.shape
    V, D = token_table.shape
    xt = x.astype(jnp.int32).T.reshape(-1)
    out = _make_kernel(B, T, D, V)(xt, token_table, pos_table)
    return out.reshape(B, T, D)


# speedup vs baseline: 7.4467x; 1.6768x over previous
import functools

import jax
import jax.numpy as jnp
from jax import lax
from jax.experimental import pallas as pl
from jax.experimental.pallas import tpu as pltpu
from jax.experimental.pallas import tpu_sc as plsc

_L = 16
_GCH = 128


@functools.lru_cache(maxsize=None)
def _make_kernel(B, T, D, V):
    NC, NS = 2, 16
    NW = NC * NS
    CH = 256
    NQ = B // CH
    DV = D // _L
    NG = CH // _GCH
    NTMAX = (T + NW - 1) // NW

    mesh = plsc.VectorSubcoreMesh(core_axis_name="c", subcore_axis_name="s")

    @functools.partial(
        pl.kernel,
        out_type=jax.ShapeDtypeStruct((B * T, D), jnp.float32),
        mesh=mesh,
        scratch_types=[
            pltpu.VMEM((NTMAX, B), jnp.int32),
            pltpu.VMEM((NTMAX, D), jnp.float32),
            pltpu.VMEM((2, CH, D), jnp.float32),
            pltpu.VMEM((2, NG, _GCH), jnp.int32),
            pltpu.SemaphoreType.DMA,
            pltpu.SemaphoreType.DMA,
            pltpu.SemaphoreType.DMA,
            pltpu.SemaphoreType.DMA,
            pltpu.SemaphoreType.DMA,
        ],
    )
    def kern(xt_hbm, tok_hbm, pos_hbm, out_hbm,
             idx_v, pos_v, rows_v, oidx_v, psem, gsem0, gsem1, ssem0, ssem1):
        cid = lax.axis_index("c")
        sid = lax.axis_index("s")
        wid = sid * NC + cid
        gsems = (gsem0, gsem1)
        ssems = (ssem0, ssem1)

        nt = (T - wid + NW - 1) // NW
        ntasks = nt * NQ

        def pref(ti, c):
            t = wid + ti * NW
            pltpu.async_copy(xt_hbm.at[pl.ds(t * B, B)], idx_v.at[ti], psem)
            pltpu.async_copy(pos_hbm.at[t], pos_v.at[ti], psem)
            return c

        def pref_wait(ti, c):
            t = wid + ti * NW
            pltpu.make_async_copy(xt_hbm.at[pl.ds(t * B, B)], idx_v.at[ti],
                                  psem).wait()
            pltpu.make_async_copy(pos_hbm.at[t], pos_v.at[ti], psem).wait()
            return c

        lax.fori_loop(0, nt, pref, 0)
        lax.fori_loop(0, nt, pref_wait, 0)

        def gather_copies(i, b, make_only):
            ti = i // NQ
            base = (i % NQ) * CH
            out = []
            for j in range(NG):
                src = tok_hbm.at[idx_v.at[ti, pl.ds(base + j * _GCH, _GCH)]]
                dst = rows_v.at[b, pl.ds(j * _GCH, _GCH)]
                if make_only:
                    out.append(pltpu.make_async_copy(src, dst, gsems[b]))
                else:
                    out.append(pltpu.async_copy(src, dst, gsems[b]))
            return out

        def scatter_copies(i, b, make_only):
            out = []
            for j in range(NG):
                src = rows_v.at[b, pl.ds(j * _GCH, _GCH)]
                dst = out_hbm.at[oidx_v.at[b, j]]
                if make_only:
                    out.append(pltpu.make_async_copy(src, dst, ssems[b]))
                else:
                    out.append(pltpu.async_copy(src, dst, ssems[b]))
            return out

        def process(i, b):
            for cp in gather_copies(i, b, True):
                cp.wait()

            ti = i // NQ
            base = (i % NQ) * CH
            t = wid + ti * NW

            posv = [pos_v[ti, pl.ds(k * _L, _L)] for k in range(DV)]

            def row(r, c):
                for k in range(DV):
                    sl = pl.ds(k * _L, _L)
                    rows_v[b, r, sl] = rows_v[b, r, sl] + posv[k]
                return c

            lax.fori_loop(0, CH, row, 0)

            lanes = lax.iota(jnp.int32, _L) * T
            for j in range(NG):
                for k in range(_GCH // _L):
                    c0 = (base + j * _GCH + k * _L) * T
                    oidx_v[b, j, pl.ds(k * _L, _L)] = lanes + (c0 + t)

            scatter_copies(i, b, False)

        gather_copies(0, 0, False)

        def group(g, c):
            i0 = 2 * g

            @pl.when(g >= 1)
            def _():
                for cp in scatter_copies(i0 - 1, 1, True):
                    cp.wait()

            gather_copies(i0 + 1, 1, False)
            process(i0, 0)

            @pl.when(i0 + 2 < ntasks)
            def _():
                for cp in scatter_copies(i0, 0, True):
                    cp.wait()
                gather_copies(i0 + 2, 0, False)

            process(i0 + 1, 1)
            return c

        lax.fori_loop(0, ntasks // 2, group, 0)

        for cp in scatter_copies(ntasks - 2, 0, True):
            cp.wait()
        for cp in scatter_copies(ntasks - 1, 1, True):
            cp.wait()

    return kern


def kernel(x, token_table, pos_table):
    B, T = x.shape
    V, D = token_table.shape
    xt = x.astype(jnp.int32).T.reshape(-1)
    out = _make_kernel(B, T, D, V)(xt, token_table, pos_table)
    return out.reshape(B, T, D)


# speedup vs baseline: 7.8954x; 1.0603x over previous
import functools

import jax
import jax.numpy as jnp
from jax import lax
from jax.experimental import pallas as pl
from jax.experimental.pallas import tpu as pltpu
from jax.experimental.pallas import tpu_sc as plsc

_L = 16
_GCH = 128


@functools.lru_cache(maxsize=None)
def _make_kernel(B, T, D, V):
    NC, NS = 2, 16
    NW = NC * NS
    CH = 256
    NQ = B // CH
    DV = D // _L
    NG = CH // _GCH
    NTASK = (T * NQ) // NW
    assert T * NQ == NTASK * NW
    PROWS = 16

    mesh = plsc.VectorSubcoreMesh(core_axis_name="c", subcore_axis_name="s")

    @functools.partial(
        pl.kernel,
        out_type=jax.ShapeDtypeStruct((B * T, D), jnp.float32),
        mesh=mesh,
        scratch_types=[
            pltpu.VMEM((NTASK * CH,), jnp.int32),
            pltpu.VMEM((PROWS, D), jnp.float32),
            pltpu.VMEM((2, CH, D), jnp.float32),
            pltpu.VMEM((2, NG, _GCH), jnp.int32),
            pltpu.SemaphoreType.DMA,
            pltpu.SemaphoreType.DMA,
            pltpu.SemaphoreType.DMA,
            pltpu.SemaphoreType.DMA,
            pltpu.SemaphoreType.DMA,
        ],
    )
    def kern(xt_hbm, tok_hbm, pos_hbm, out_hbm,
             idx_v, pos_v, rows_v, oidx_v, psem, gsem0, gsem1, ssem0, ssem1):
        cid = lax.axis_index("c")
        sid = lax.axis_index("s")
        wid = sid * NC + cid
        gsems = (gsem0, gsem1)
        ssems = (ssem0, ssem1)

        tau0 = wid * NTASK
        tlo = tau0 // NQ
        pstart = jnp.minimum((tlo // 8) * 8, T - PROWS)

        cp_i = pltpu.async_copy(
            xt_hbm.at[pl.ds(tau0 * CH, NTASK * CH)], idx_v, psem)
        cp_p = pltpu.async_copy(
            pos_hbm.at[pl.ds(pstart, PROWS)], pos_v, psem)
        cp_i.wait()
        cp_p.wait()

        def gather_copies(j, b, make_only):
            out = []
            for jj in range(NG):
                src = tok_hbm.at[
                    idx_v.at[pl.ds(j * CH + jj * _GCH, _GCH)]]
                dst = rows_v.at[b, pl.ds(jj * _GCH, _GCH)]
                if make_only:
                    out.append(pltpu.make_async_copy(src, dst, gsems[b]))
                else:
                    out.append(pltpu.async_copy(src, dst, gsems[b]))
            return out

        def scatter_copies(b, make_only):
            out = []
            for jj in range(NG):
                src = rows_v.at[b, pl.ds(jj * _GCH, _GCH)]
                dst = out_hbm.at[oidx_v.at[b, jj]]
                if make_only:
                    out.append(pltpu.make_async_copy(src, dst, ssems[b]))
                else:
                    out.append(pltpu.async_copy(src, dst, ssems[b]))
            return out

        def process(j, b):
            for cp in gather_copies(j, b, True):
                cp.wait()

            tau = tau0 + j
            t = tau // NQ
            base = (tau % NQ) * CH
            t_local = t - pstart

            posv = [pos_v[t_local, pl.ds(k * _L, _L)] for k in range(DV)]

            def row(r, c):
                for k in range(DV):
                    sl = pl.ds(k * _L, _L)
                    rows_v[b, r, sl] = rows_v[b, r, sl] + posv[k]
                return c

            lax.fori_loop(0, CH, row, 0)

            lanes = lax.iota(jnp.int32, _L) * T
            for jj in range(NG):
                for k in range(_GCH // _L):
                    c0 = (base + jj * _GCH + k * _L) * T
                    oidx_v[b, jj, pl.ds(k * _L, _L)] = lanes + (c0 + t)

            scatter_copies(b, False)

        gather_copies(0, 0, False)

        def group(g, c):
            j0 = 2 * g

            @pl.when(g >= 1)
            def _():
                for cp in scatter_copies(1, True):
                    cp.wait()

            gather_copies(j0 + 1, 1, False)
            process(j0, 0)

            for cp in scatter_copies(0, True):
                cp.wait()
            gather_copies(j0 + 2, 0, False)

            process(j0 + 1, 1)
            return c

        lax.fori_loop(0, (NTASK - 1) // 2, group, 0)

        process(NTASK - 1, 0)
        for cp in scatter_copies(1, True):
            cp.wait()
        for cp in scatter_copies(0, True):
            cp.wait()

    return kern


def kernel(x, token_table, pos_table):
    B, T = x.shape
    V, D = token_table.shape
    xt = x.astype(jnp.int32).T.reshape(-1)
    out = _make_kernel(B, T, D, V)(xt, token_table, pos_table)
    return out.reshape(B, T, D)
